# baseline (speedup 1.0000x reference)
import dataclasses
import functools

import jax
import jax.numpy as jnp
from jax import lax
from jax.experimental import pallas as pl
from jax.experimental.pallas import tpu as pltpu
from jax.experimental.pallas import tpu_sc as plsc

N_NODES = 10000
N_EDGES = 320000
D_IN = 128
N_CLASSES = 64

NC = 2
NS = 16
L = 16
NW = NC * NS

NPAD = 10240
K = 128
EPT = 10240
EPAD = NW * EPT
NCHUNK = EPT // K
ROWS_PT = NPAD // NS

_MESH = plsc.VectorSubcoreMesh(
    core_axis_name="c", subcore_axis_name="s", num_cores=NC, num_subcores=NS
)


def _make_scatter(d, ks, nslot, cp=None):
    nchunk = EPT // ks
    assert nchunk % nslot == 0 and nslot >= 4

    @functools.partial(
        pl.kernel,
        out_type=jax.ShapeDtypeStruct((NC, NPAD, d), jnp.float32),
        mesh=_MESH,
        compiler_params=cp,
        scratch_types=(
            [pltpu.VMEM((EPT,), jnp.int32)]
            + [pltpu.VMEM((ks,), jnp.int32) for _ in range(nslot)]
            + [pltpu.VMEM((ks, d), jnp.float32) for _ in range(nslot)]
            + [pltpu.VMEM_SHARED((NPAD, d), jnp.float32)]
            + [pltpu.SemaphoreType.DMA for _ in range(3 * nslot)]
        ),
    )
    def scatter_kernel(xp_hbm, srcp_hbm, dstp_hbm, out_hbm, src_v, *rest):
        dstb = rest[0:nslot]
        rows = rest[nslot:2 * nslot]
        acc_sh = rest[2 * nslot]
        isem = rest[2 * nslot + 1:3 * nslot + 1]
        gsem = rest[3 * nslot + 1:4 * nslot + 1]
        ssem = rest[4 * nslot + 1:5 * nslot + 1]
        cid = lax.axis_index("c")
        sid = lax.axis_index("s")
        wid = cid * NS + sid
        r0 = rows[0]

        ebase = wid * EPT
        i_src = pltpu.async_copy(
            srcp_hbm.at[pl.ds(ebase, EPT)], src_v, gsem[0])
        i_src.wait()

        def fetch(cc, b):
            pltpu.async_copy(
                dstp_hbm.at[pl.ds(ebase + cc * ks, ks)], dstb[b], isem[b])
            pltpu.async_copy(
                xp_hbm.at[src_v.at[pl.ds(cc * ks, ks)]], rows[b], gsem[b])

        def wait_fetch(b):
            pltpu.make_async_copy(
                dstp_hbm.at[pl.ds(ebase, ks)], dstb[b], isem[b]).wait()
            pltpu.make_async_copy(
                xp_hbm.at[src_v.at[pl.ds(0, ks)]], rows[b], gsem[b]).wait()

        def scat(b):
            pltpu.async_copy(rows[b], acc_sh.at[dstb[b]], ssem[b], add=True)

        def wait_scat(b):
            pltpu.make_async_copy(rows[b], acc_sh.at[dstb[b]], ssem[b]).wait()

        for cc in range(1, nslot):
            fetch(cc, cc)

        zeros = jnp.zeros((L,), jnp.float32)

        @pl.loop(0, ks)
        def _(r):
            @pl.loop(0, d // L)
            def _(j):
                r0[r, pl.ds(j * L, L)] = zeros

        rbase = sid * ROWS_PT
        for j in range(ROWS_PT // ks):
            pltpu.async_copy(
                r0, acc_sh.at[pl.ds(rbase + j * ks, ks)], ssem[0])
        for j in range(ROWS_PT // ks):
            pltpu.make_async_copy(
                r0, acc_sh.at[pl.ds(rbase, ks)], ssem[0]).wait()
        fetch(0, 0)
        plsc.subcore_barrier()

        wait_fetch(0)
        scat(0)
        wait_fetch(1)
        scat(1)

        @pl.loop(2, 2 + (nchunk - nslot), step=nslot)
        def _(c):
            for j in range(nslot):
                b = (2 + j) % nslot
                b2 = j
                wait_fetch(b)
                wait_scat(b2)
                fetch_c = c + j + (nslot - 2)
                pltpu.async_copy(
                    dstp_hbm.at[pl.ds(ebase + fetch_c * ks, ks)],
                    dstb[b2], isem[b2])
                pltpu.async_copy(
                    xp_hbm.at[src_v.at[pl.ds(fetch_c * ks, ks)]],
                    rows[b2], gsem[b2])
                scat(b)

        for cc in range(nchunk - nslot + 2, nchunk):
            wait_fetch(cc % nslot)
            wait_scat((cc - 2) % nslot)
            scat(cc % nslot)
        for cc in (nchunk - 2, nchunk - 1):
            wait_scat(cc % nslot)

        plsc.subcore_barrier()
        pltpu.sync_copy(acc_sh.at[pl.ds(rbase, ROWS_PT)],
                        out_hbm.at[cid, pl.ds(rbase, ROWS_PT)])

    return scatter_kernel


_CP_LINEAR = dataclasses.replace(pltpu.CompilerParams(),
                                 use_tc_tiling_on_sc=False)

_scatter128 = _make_scatter(D_IN, 32, 8)
_scatter64 = _make_scatter(N_CLASSES, 64, 10, cp=_CP_LINEAR)


_CP = pltpu.CompilerParams()
if "needs_layout_passes" in pltpu.CompilerParams.__dataclass_fields__:
    _CP = dataclasses.replace(_CP, needs_layout_passes=False)


@functools.partial(
    pl.kernel,
    out_type=(
        jax.ShapeDtypeStruct((NW, NPAD), jnp.float32),
        jax.ShapeDtypeStruct((NW, NPAD), jnp.float32),
    ),
    mesh=_MESH,
    compiler_params=_CP,
    scratch_types=[
        pltpu.VMEM((NPAD,), jnp.float32),
        pltpu.VMEM((NPAD,), jnp.float32),
        pltpu.VMEM((EPT,), jnp.int32),
        pltpu.VMEM((EPT,), jnp.int32),
        pltpu.SemaphoreType.DMA,
        pltpu.SemaphoreType.DMA,
    ],
)
def _deg_kernel(srcp_hbm, dstp_hbm, osrc_hbm, odst_hbm, hs_v, hd_v,
                src_v, dst_v, sem_a, sem_b):
    cid = lax.axis_index("c")
    sid = lax.axis_index("s")
    wid = cid * NS + sid

    ebase = wid * EPT
    i_src = pltpu.async_copy(srcp_hbm.at[pl.ds(ebase, EPT)], src_v, sem_a)
    i_dst = pltpu.async_copy(dstp_hbm.at[pl.ds(ebase, EPT)], dst_v, sem_b)

    zeros = jnp.zeros((L,), jnp.float32)

    @pl.loop(0, NPAD // L)
    def _(i):
        hs_v[pl.ds(i * L, L)] = zeros
        hd_v[pl.ds(i * L, L)] = zeros

    ones = jnp.ones((L,), jnp.float32)
    i_src.wait()
    i_dst.wait()

    @pl.loop(0, EPT // L)
    def _(i):
        sv = src_v[pl.ds(i * L, L)]
        dv = dst_v[pl.ds(i * L, L)]
        plsc.addupdate_scatter(hs_v, [sv], ones)
        plsc.addupdate_scatter(hd_v, [dv], ones)

    pltpu.sync_copy(hs_v, osrc_hbm.at[wid])
    pltpu.sync_copy(hd_v, odst_hbm.at[wid])


def _prologue_body(x_ref, dsp_ref, ddp_ref, xp_ref, ns_ref, nd_ref):
    deg_out = jnp.sum(dsp_ref[...], axis=0)
    deg_in = jnp.sum(ddp_ref[...], axis=0)
    ns = lax.rsqrt(jnp.clip(deg_out, 1.0, None))
    nd = lax.rsqrt(jnp.clip(deg_in, 1.0, None))
    ns_ref[...] = ns[:, None]
    nd_ref[...] = nd[:, None]
    xs = x_ref[...] * ns[:N_NODES, None]
    xp_ref[...] = jnp.concatenate(
        [xs, jnp.zeros((NPAD - N_NODES, D_IN), jnp.float32)], axis=0)


_prologue = pl.pallas_call(
    _prologue_body,
    out_shape=(
        jax.ShapeDtypeStruct((NPAD, D_IN), jnp.float32),
        jax.ShapeDtypeStruct((NPAD, 1), jnp.float32),
        jax.ShapeDtypeStruct((NPAD, 1), jnp.float32),
    ),
)


def _mid_body(a_ref, nd_ref, ns_ref, w1_ref, b1_ref, w2_ref, g_ref):
    agg = a_ref[0] + a_ref[1]
    h = jnp.dot(agg * nd_ref[...], w1_ref[...],
                preferred_element_type=jnp.float32)
    h = jnp.maximum(h + b1_ref[...], 0.0)
    g_ref[...] = jnp.dot(h * ns_ref[...], w2_ref[...],
                         preferred_element_type=jnp.float32)


_mid = pl.pallas_call(
    _mid_body,
    out_shape=jax.ShapeDtypeStruct((NPAD, N_CLASSES), jnp.float32),
)


def _final_body(a_ref, nd_ref, b2_ref, o_ref):
    agg = a_ref[0] + a_ref[1]
    o_ref[...] = jnp.maximum(
        agg[:N_NODES] * nd_ref[:N_NODES] + b2_ref[...], 0.0)


_final = pl.pallas_call(
    _final_body,
    out_shape=jax.ShapeDtypeStruct((N_NODES, N_CLASSES), jnp.float32),
)


def kernel(x, edge_index, W1, b1, W2, b2):
    src = edge_index[0].astype(jnp.int32)
    dst = edge_index[1].astype(jnp.int32)
    pad = N_NODES + jnp.arange(EPAD - N_EDGES, dtype=jnp.int32) % (NPAD - N_NODES)
    srcp = jnp.concatenate([src, pad])
    dstp = jnp.concatenate([dst, pad])
    dsp, ddp = _deg_kernel(srcp, dstp)
    xp, ns, nd = _prologue(x, dsp, ddp)
    agg1 = _scatter128(xp, srcp, dstp)
    g = _mid(agg1, nd, ns, W1, b1, W2)
    agg2 = _scatter64(g, srcp, dstp)
    return _final(agg2, nd, b2)


# speedup vs baseline: 1.0129x; 1.0129x over previous
import dataclasses
import functools

import jax
import jax.numpy as jnp
from jax import lax
from jax.experimental import pallas as pl
from jax.experimental.pallas import tpu as pltpu
from jax.experimental.pallas import tpu_sc as plsc

N_NODES = 10000
N_EDGES = 320000
D_IN = 128
N_CLASSES = 64

NC = 2
NS = 16
L = 16
NW = NC * NS

NPAD = 10240
K = 128
EPT = 10240
EPAD = NW * EPT
NCHUNK = EPT // K
ROWS_PT = NPAD // NS

_MESH = plsc.VectorSubcoreMesh(
    core_axis_name="c", subcore_axis_name="s", num_cores=NC, num_subcores=NS
)


def _make_scatter(d, ks, nslot, cp=None):
    nchunk = EPT // ks
    assert nchunk % nslot == 0 and nslot >= 4

    @functools.partial(
        pl.kernel,
        out_type=jax.ShapeDtypeStruct((NC, NPAD, d), jnp.float32),
        mesh=_MESH,
        compiler_params=cp,
        scratch_types=(
            [pltpu.VMEM((EPT,), jnp.int32)]
            + [pltpu.VMEM((ks,), jnp.int32) for _ in range(nslot)]
            + [pltpu.VMEM((ks, d), jnp.float32) for _ in range(nslot)]
            + [pltpu.VMEM_SHARED((NPAD, d), jnp.float32)]
            + [pltpu.SemaphoreType.DMA for _ in range(3 * nslot)]
        ),
    )
    def scatter_kernel(xp_hbm, srcp_hbm, dstp_hbm, out_hbm, src_v, *rest):
        dstb = rest[0:nslot]
        rows = rest[nslot:2 * nslot]
        acc_sh = rest[2 * nslot]
        isem = rest[2 * nslot + 1:3 * nslot + 1]
        gsem = rest[3 * nslot + 1:4 * nslot + 1]
        ssem = rest[4 * nslot + 1:5 * nslot + 1]
        cid = lax.axis_index("c")
        sid = lax.axis_index("s")
        wid = cid * NS + sid
        r0 = rows[0]

        ebase = wid * EPT
        i_src = pltpu.async_copy(
            srcp_hbm.at[pl.ds(ebase, EPT)], src_v, gsem[0])
        i_src.wait()

        def fetch(cc, b):
            pltpu.async_copy(
                dstp_hbm.at[pl.ds(ebase + cc * ks, ks)], dstb[b], isem[b])
            pltpu.async_copy(
                xp_hbm.at[src_v.at[pl.ds(cc * ks, ks)]], rows[b], gsem[b])

        def wait_fetch(b):
            pltpu.make_async_copy(
                dstp_hbm.at[pl.ds(ebase, ks)], dstb[b], isem[b]).wait()
            pltpu.make_async_copy(
                xp_hbm.at[src_v.at[pl.ds(0, ks)]], rows[b], gsem[b]).wait()

        def scat(b):
            pltpu.async_copy(rows[b], acc_sh.at[dstb[b]], ssem[b], add=True)

        def wait_scat(b):
            pltpu.make_async_copy(rows[b], acc_sh.at[dstb[b]], ssem[b]).wait()

        for cc in range(1, nslot):
            fetch(cc, cc)

        zeros = jnp.zeros((L,), jnp.float32)

        @pl.loop(0, ks)
        def _(r):
            @pl.loop(0, d // L)
            def _(j):
                r0[r, pl.ds(j * L, L)] = zeros

        rbase = sid * ROWS_PT
        for j in range(ROWS_PT // ks):
            pltpu.async_copy(
                r0, acc_sh.at[pl.ds(rbase + j * ks, ks)], ssem[0])
        for j in range(ROWS_PT // ks):
            pltpu.make_async_copy(
                r0, acc_sh.at[pl.ds(rbase, ks)], ssem[0]).wait()
        fetch(0, 0)
        plsc.subcore_barrier()

        wait_fetch(0)
        scat(0)
        wait_fetch(1)
        scat(1)

        @pl.loop(2, 2 + (nchunk - nslot), step=nslot)
        def _(c):
            for j in range(nslot):
                b = (2 + j) % nslot
                b2 = j
                wait_fetch(b)
                wait_scat(b2)
                fetch_c = c + j + (nslot - 2)
                pltpu.async_copy(
                    dstp_hbm.at[pl.ds(ebase + fetch_c * ks, ks)],
                    dstb[b2], isem[b2])
                pltpu.async_copy(
                    xp_hbm.at[src_v.at[pl.ds(fetch_c * ks, ks)]],
                    rows[b2], gsem[b2])
                scat(b)

        for cc in range(nchunk - nslot + 2, nchunk):
            wait_fetch(cc % nslot)
            wait_scat((cc - 2) % nslot)
            scat(cc % nslot)
        for cc in (nchunk - 2, nchunk - 1):
            wait_scat(cc % nslot)

        plsc.subcore_barrier()
        pltpu.sync_copy(acc_sh.at[pl.ds(rbase, ROWS_PT)],
                        out_hbm.at[cid, pl.ds(rbase, ROWS_PT)])

    return scatter_kernel


_CP_LINEAR = dataclasses.replace(pltpu.CompilerParams(),
                                 use_tc_tiling_on_sc=False)

_scatter128 = _make_scatter(D_IN, 32, 8)
_scatter64 = _make_scatter(N_CLASSES, 128, 8, cp=_CP_LINEAR)


_CP = pltpu.CompilerParams()
if "needs_layout_passes" in pltpu.CompilerParams.__dataclass_fields__:
    _CP = dataclasses.replace(_CP, needs_layout_passes=False)


@functools.partial(
    pl.kernel,
    out_type=(
        jax.ShapeDtypeStruct((NW, NPAD), jnp.float32),
        jax.ShapeDtypeStruct((NW, NPAD), jnp.float32),
    ),
    mesh=_MESH,
    compiler_params=_CP,
    scratch_types=[
        pltpu.VMEM((NPAD,), jnp.float32),
        pltpu.VMEM((NPAD,), jnp.float32),
        pltpu.VMEM((EPT,), jnp.int32),
        pltpu.VMEM((EPT,), jnp.int32),
        pltpu.SemaphoreType.DMA,
        pltpu.SemaphoreType.DMA,
    ],
)
def _deg_kernel(srcp_hbm, dstp_hbm, osrc_hbm, odst_hbm, hs_v, hd_v,
                src_v, dst_v, sem_a, sem_b):
    cid = lax.axis_index("c")
    sid = lax.axis_index("s")
    wid = cid * NS + sid

    ebase = wid * EPT
    i_src = pltpu.async_copy(srcp_hbm.at[pl.ds(ebase, EPT)], src_v, sem_a)
    i_dst = pltpu.async_copy(dstp_hbm.at[pl.ds(ebase, EPT)], dst_v, sem_b)

    zeros = jnp.zeros((L,), jnp.float32)

    @pl.loop(0, NPAD // L)
    def _(i):
        hs_v[pl.ds(i * L, L)] = zeros
        hd_v[pl.ds(i * L, L)] = zeros

    ones = jnp.ones((L,), jnp.float32)
    i_src.wait()
    i_dst.wait()

    @pl.loop(0, EPT // L)
    def _(i):
        sv = src_v[pl.ds(i * L, L)]
        dv = dst_v[pl.ds(i * L, L)]
        plsc.addupdate_scatter(hs_v, [sv], ones)
        plsc.addupdate_scatter(hd_v, [dv], ones)

    pltpu.sync_copy(hs_v, osrc_hbm.at[wid])
    pltpu.sync_copy(hd_v, odst_hbm.at[wid])


def _prologue_body(x_ref, dsp_ref, ddp_ref, xp_ref, ns_ref, nd_ref):
    deg_out = jnp.sum(dsp_ref[...], axis=0)
    deg_in = jnp.sum(ddp_ref[...], axis=0)
    ns = lax.rsqrt(jnp.clip(deg_out, 1.0, None))
    nd = lax.rsqrt(jnp.clip(deg_in, 1.0, None))
    ns_ref[...] = ns[:, None]
    nd_ref[...] = nd[:, None]
    xs = x_ref[...] * ns[:N_NODES, None]
    xp_ref[...] = jnp.concatenate(
        [xs, jnp.zeros((NPAD - N_NODES, D_IN), jnp.float32)], axis=0)


_prologue = pl.pallas_call(
    _prologue_body,
    out_shape=(
        jax.ShapeDtypeStruct((NPAD, D_IN), jnp.float32),
        jax.ShapeDtypeStruct((NPAD, 1), jnp.float32),
        jax.ShapeDtypeStruct((NPAD, 1), jnp.float32),
    ),
)


def _mid_body(a_ref, nd_ref, ns_ref, w1_ref, b1_ref, w2_ref, g_ref):
    agg = a_ref[0] + a_ref[1]
    h = jnp.dot(agg * nd_ref[...], w1_ref[...],
                preferred_element_type=jnp.float32)
    h = jnp.maximum(h + b1_ref[...], 0.0)
    g_ref[...] = jnp.dot(h * ns_ref[...], w2_ref[...],
                         preferred_element_type=jnp.float32)


_mid = pl.pallas_call(
    _mid_body,
    out_shape=jax.ShapeDtypeStruct((NPAD, N_CLASSES), jnp.float32),
)


def _final_body(a_ref, nd_ref, b2_ref, o_ref):
    agg = a_ref[0] + a_ref[1]
    o_ref[...] = jnp.maximum(
        agg[:N_NODES] * nd_ref[:N_NODES] + b2_ref[...], 0.0)


_final = pl.pallas_call(
    _final_body,
    out_shape=jax.ShapeDtypeStruct((N_NODES, N_CLASSES), jnp.float32),
)


def kernel(x, edge_index, W1, b1, W2, b2):
    src = edge_index[0].astype(jnp.int32)
    dst = edge_index[1].astype(jnp.int32)
    pad = N_NODES + jnp.arange(EPAD - N_EDGES, dtype=jnp.int32) % (NPAD - N_NODES)
    srcp = jnp.concatenate([src, pad])
    dstp = jnp.concatenate([dst, pad])
    dsp, ddp = _deg_kernel(srcp, dstp)
    xp, ns, nd = _prologue(x, dsp, ddp)
    agg1 = _scatter128(xp, srcp, dstp)
    g = _mid(agg1, nd, ns, W1, b1, W2)
    agg2 = _scatter64(g, srcp, dstp)
    return _final(agg2, nd, b2)


# speedup vs baseline: 1.0310x; 1.0179x over previous
import dataclasses
import functools

import jax
import jax.numpy as jnp
from jax import lax
from jax.experimental import pallas as pl
from jax.experimental.pallas import tpu as pltpu
from jax.experimental.pallas import tpu_sc as plsc

N_NODES = 10000
N_EDGES = 320000
D_IN = 128
N_CLASSES = 64

NC = 2
NS = 16
L = 16
NW = NC * NS

NPAD = 10240
K = 128
EPT = 10240
EPAD = NW * EPT
NCHUNK = EPT // K
ROWS_PT = NPAD // NS

_MESH = plsc.VectorSubcoreMesh(
    core_axis_name="c", subcore_axis_name="s", num_cores=NC, num_subcores=NS
)


def _make_scatter(d, ks, nslot, cp=None):
    nchunk = EPT // ks
    assert nchunk % nslot == 0 and nslot >= 4

    @functools.partial(
        pl.kernel,
        out_type=jax.ShapeDtypeStruct((NC, NPAD, d), jnp.float32),
        mesh=_MESH,
        compiler_params=cp,
        scratch_types=(
            [pltpu.VMEM((EPT,), jnp.int32)]
            + [pltpu.VMEM((ks,), jnp.int32) for _ in range(nslot)]
            + [pltpu.VMEM((ks, d), jnp.float32) for _ in range(nslot)]
            + [pltpu.VMEM_SHARED((NPAD, d), jnp.float32)]
            + [pltpu.SemaphoreType.DMA for _ in range(3 * nslot)]
        ),
    )
    def scatter_kernel(xp_hbm, ep_hbm, out_hbm, src_v, *rest):
        dstb = rest[0:nslot]
        rows = rest[nslot:2 * nslot]
        acc_sh = rest[2 * nslot]
        isem = rest[2 * nslot + 1:3 * nslot + 1]
        gsem = rest[3 * nslot + 1:4 * nslot + 1]
        ssem = rest[4 * nslot + 1:5 * nslot + 1]
        cid = lax.axis_index("c")
        sid = lax.axis_index("s")
        wid = cid * NS + sid
        r0 = rows[0]

        ebase = wid * EPT
        i_src = pltpu.async_copy(
            ep_hbm.at[0, pl.ds(ebase, EPT)], src_v, gsem[0])
        i_src.wait()

        def fetch(cc, b):
            pltpu.async_copy(
                ep_hbm.at[1, pl.ds(ebase + cc * ks, ks)], dstb[b], isem[b])
            pltpu.async_copy(
                xp_hbm.at[src_v.at[pl.ds(cc * ks, ks)]], rows[b], gsem[b])

        def wait_fetch(b):
            pltpu.make_async_copy(
                ep_hbm.at[1, pl.ds(ebase, ks)], dstb[b], isem[b]).wait()
            pltpu.make_async_copy(
                xp_hbm.at[src_v.at[pl.ds(0, ks)]], rows[b], gsem[b]).wait()

        def scat(b):
            pltpu.async_copy(rows[b], acc_sh.at[dstb[b]], ssem[b], add=True)

        def wait_scat(b):
            pltpu.make_async_copy(rows[b], acc_sh.at[dstb[b]], ssem[b]).wait()

        for cc in range(1, nslot):
            fetch(cc, cc)

        zeros = jnp.zeros((L,), jnp.float32)

        @pl.loop(0, ks)
        def _(r):
            @pl.loop(0, d // L)
            def _(j):
                r0[r, pl.ds(j * L, L)] = zeros

        rbase = sid * ROWS_PT
        for j in range(ROWS_PT // ks):
            pltpu.async_copy(
                r0, acc_sh.at[pl.ds(rbase + j * ks, ks)], ssem[0])
        for j in range(ROWS_PT // ks):
            pltpu.make_async_copy(
                r0, acc_sh.at[pl.ds(rbase, ks)], ssem[0]).wait()
        fetch(0, 0)
        plsc.subcore_barrier()

        wait_fetch(0)
        scat(0)
        wait_fetch(1)
        scat(1)

        @pl.loop(2, 2 + (nchunk - nslot), step=nslot)
        def _(c):
            for j in range(nslot):
                b = (2 + j) % nslot
                b2 = j
                wait_fetch(b)
                wait_scat(b2)
                fetch_c = c + j + (nslot - 2)
                pltpu.async_copy(
                    ep_hbm.at[1, pl.ds(ebase + fetch_c * ks, ks)],
                    dstb[b2], isem[b2])
                pltpu.async_copy(
                    xp_hbm.at[src_v.at[pl.ds(fetch_c * ks, ks)]],
                    rows[b2], gsem[b2])
                scat(b)

        for cc in range(nchunk - nslot + 2, nchunk):
            wait_fetch(cc % nslot)
            wait_scat((cc - 2) % nslot)
            scat(cc % nslot)
        for cc in (nchunk - 2, nchunk - 1):
            wait_scat(cc % nslot)

        plsc.subcore_barrier()
        pltpu.sync_copy(acc_sh.at[pl.ds(rbase, ROWS_PT)],
                        out_hbm.at[cid, pl.ds(rbase, ROWS_PT)])

    return scatter_kernel


_CP_LINEAR = dataclasses.replace(pltpu.CompilerParams(),
                                 use_tc_tiling_on_sc=False)

_scatter128 = _make_scatter(D_IN, 32, 8)
_scatter64 = _make_scatter(N_CLASSES, 128, 8, cp=_CP_LINEAR)


_CP = pltpu.CompilerParams()
if "needs_layout_passes" in pltpu.CompilerParams.__dataclass_fields__:
    _CP = dataclasses.replace(_CP, needs_layout_passes=False)


@functools.partial(
    pl.kernel,
    out_type=(
        jax.ShapeDtypeStruct((NW, NPAD), jnp.float32),
        jax.ShapeDtypeStruct((NW, NPAD), jnp.float32),
    ),
    mesh=_MESH,
    compiler_params=_CP,
    scratch_types=[
        pltpu.VMEM((NPAD,), jnp.float32),
        pltpu.VMEM((NPAD,), jnp.float32),
        pltpu.VMEM((EPT,), jnp.int32),
        pltpu.VMEM((EPT,), jnp.int32),
        pltpu.SemaphoreType.DMA,
        pltpu.SemaphoreType.DMA,
    ],
)
def _deg_kernel(ep_hbm, osrc_hbm, odst_hbm, hs_v, hd_v,
                src_v, dst_v, sem_a, sem_b):
    cid = lax.axis_index("c")
    sid = lax.axis_index("s")
    wid = cid * NS + sid

    ebase = wid * EPT
    i_src = pltpu.async_copy(ep_hbm.at[0, pl.ds(ebase, EPT)], src_v, sem_a)
    i_dst = pltpu.async_copy(ep_hbm.at[1, pl.ds(ebase, EPT)], dst_v, sem_b)

    zeros = jnp.zeros((L,), jnp.float32)

    @pl.loop(0, NPAD // L)
    def _(i):
        hs_v[pl.ds(i * L, L)] = zeros
        hd_v[pl.ds(i * L, L)] = zeros

    ones = jnp.ones((L,), jnp.float32)
    i_src.wait()
    i_dst.wait()

    @pl.loop(0, EPT // L)
    def _(i):
        sv = src_v[pl.ds(i * L, L)]
        dv = dst_v[pl.ds(i * L, L)]
        plsc.addupdate_scatter(hs_v, [sv], ones)
        plsc.addupdate_scatter(hd_v, [dv], ones)

    pltpu.sync_copy(hs_v, osrc_hbm.at[wid])
    pltpu.sync_copy(hd_v, odst_hbm.at[wid])


def _prologue_body(x_ref, dsp_ref, ddp_ref, xp_ref, ns_ref, nd_ref):
    deg_out = jnp.sum(dsp_ref[...], axis=0)
    deg_in = jnp.sum(ddp_ref[...], axis=0)
    ns = lax.rsqrt(jnp.clip(deg_out, 1.0, None))
    nd = lax.rsqrt(jnp.clip(deg_in, 1.0, None))
    ns_ref[...] = ns[:, None]
    nd_ref[...] = nd[:, None]
    xs = x_ref[...] * ns[:N_NODES, None]
    xp_ref[...] = jnp.concatenate(
        [xs, jnp.zeros((NPAD - N_NODES, D_IN), jnp.float32)], axis=0)


_prologue = pl.pallas_call(
    _prologue_body,
    out_shape=(
        jax.ShapeDtypeStruct((NPAD, D_IN), jnp.float32),
        jax.ShapeDtypeStruct((NPAD, 1), jnp.float32),
        jax.ShapeDtypeStruct((NPAD, 1), jnp.float32),
    ),
)


def _mid_body(a_ref, nd_ref, ns_ref, w1_ref, b1_ref, w2_ref, g_ref):
    agg = a_ref[0] + a_ref[1]
    h = jnp.dot(agg * nd_ref[...], w1_ref[...],
                preferred_element_type=jnp.float32)
    h = jnp.maximum(h + b1_ref[...], 0.0)
    g_ref[...] = jnp.dot(h * ns_ref[...], w2_ref[...],
                         preferred_element_type=jnp.float32)


_mid = pl.pallas_call(
    _mid_body,
    out_shape=jax.ShapeDtypeStruct((NPAD, N_CLASSES), jnp.float32),
)


def _final_body(a_ref, nd_ref, b2_ref, o_ref):
    agg = a_ref[0] + a_ref[1]
    o_ref[...] = jnp.maximum(
        agg[:N_NODES] * nd_ref[:N_NODES] + b2_ref[...], 0.0)


_final = pl.pallas_call(
    _final_body,
    out_shape=jax.ShapeDtypeStruct((N_NODES, N_CLASSES), jnp.float32),
)


def kernel(x, edge_index, W1, b1, W2, b2):
    pad = N_NODES + jnp.arange(EPAD - N_EDGES, dtype=jnp.int32) % (NPAD - N_NODES)
    ep = jnp.concatenate(
        [edge_index.astype(jnp.int32),
         jnp.broadcast_to(pad, (2, EPAD - N_EDGES))], axis=1)

    dsp, ddp = _deg_kernel(ep)
    xp, ns, nd = _prologue(x, dsp, ddp)
    agg1 = _scatter128(xp, ep)
    g = _mid(agg1, nd, ns, W1, b1, W2)
    agg2 = _scatter64(g, ep)
    return _final(agg2, nd, b2)
